# baseline (speedup 1.0000x reference)
import functools

import jax
import jax.numpy as jnp
from jax import lax
from jax.experimental import pallas as pl
from jax.experimental.pallas import tpu as pltpu
from jax.experimental.pallas import tpu_sc as plsc

B = 4096
EMBED = 64
NCAT = 8

_info = plsc.get_sparse_core_info()
_NC, _NS = _info.num_cores, _info.num_subcores
_NW = _NC * _NS
_BPW = B // _NW


def _make_kernel():
    mesh = plsc.VectorSubcoreMesh(core_axis_name="c", subcore_axis_name="s")

    @functools.partial(
        pl.kernel,
        mesh=mesh,
        out_type=jax.ShapeDtypeStruct((B, NCAT + 1, EMBED), jnp.float32),
        scratch_types=[
            pltpu.VMEM((NCAT, _BPW), jnp.int32),
            pltpu.VMEM((NCAT + 1, _BPW, EMBED), jnp.float32),
            pltpu.SemaphoreType.DMA,
        ],
        compiler_params=pltpu.CompilerParams(use_tc_tiling_on_sc=False),
    )
    def body(cat_0, cat_1, cat_2, cat_3, cat_4, cat_5, cat_6, cat_7,
             dense, table_0, table_1, table_2, table_3, table_4, table_5,
             table_6, table_7, out, idx_v, rows_v, sem):
        cats = [cat_0, cat_1, cat_2, cat_3, cat_4, cat_5, cat_6, cat_7]
        tables = [table_0, table_1, table_2, table_3, table_4, table_5,
                  table_6, table_7]
        wid = lax.axis_index("s") * _NC + lax.axis_index("c")
        base = wid * _BPW

        for i in range(NCAT):
            pltpu.sync_copy(cats[i].at[pl.ds(base, _BPW)], idx_v.at[i])

        copies = []
        for i in range(NCAT):
            copies.append(
                pltpu.async_copy(tables[i].at[idx_v.at[i]], rows_v.at[i], sem))
        copies.append(
            pltpu.async_copy(dense.at[pl.ds(base, _BPW)], rows_v.at[NCAT], sem))
        for c in copies:
            c.wait()

        for i in range(NCAT + 1):
            pltpu.sync_copy(rows_v.at[i], out.at[pl.ds(base, _BPW), i])

    return body


_kernel_call = _make_kernel()


def kernel(cat_0, cat_1, cat_2, cat_3, cat_4, cat_5, cat_6, cat_7, dense,
           table_0, table_1, table_2, table_3, table_4, table_5, table_6,
           table_7):
    out3 = _kernel_call(cat_0, cat_1, cat_2, cat_3, cat_4, cat_5, cat_6,
                        cat_7, dense, table_0, table_1, table_2, table_3,
                        table_4, table_5, table_6, table_7)
    return out3.reshape(B, (NCAT + 1) * EMBED)
